# baseline (speedup 1.0000x reference)
import dataclasses
import functools

import jax
import jax.numpy as jnp
from jax import lax
from jax.experimental import pallas as pl
from jax.experimental.pallas import tpu as pltpu
from jax.experimental.pallas import tpu_sc as plsc

N_ROWS = 50000
D = 256
G = 512
NUM_LABELS = 10

NR = 16
NCOL = 2
CW = D // NCOL

CHUNK = 400
N_CHUNKS = N_ROWS // CHUNK
CPW = 5
SC_ROWS = NR * CPW * CHUNK

TAIL = N_ROWS - SC_ROWS
BT = 1128
TB = 16
TAIL_PAD = BT * TB - TAIL
assert TAIL_PAD >= 0


def _sc_segment_sum(x, idx):
    mesh = plsc.VectorSubcoreMesh(core_axis_name="c", subcore_axis_name="s")

    cp = pltpu.CompilerParams()
    if "needs_layout_passes" in pltpu.CompilerParams.__dataclass_fields__:
        cp = dataclasses.replace(cp, needs_layout_passes=False)

    @functools.partial(
        pl.kernel,
        compiler_params=cp,
        out_type=jax.ShapeDtypeStruct((NR, G, D), jnp.float32),
        mesh=mesh,
        scratch_types=[
            pltpu.VMEM((CHUNK, CW), jnp.float32),
            pltpu.VMEM((CPW * CHUNK,), jnp.int32),
            pltpu.VMEM((G, CW), jnp.float32),
        ],
    )
    def k(x_hbm, idx_hbm, out_hbm, rows_v, idx_v, acc_v):
        c = lax.axis_index("c")
        s = lax.axis_index("s")
        wid = s * 2 + c
        r = wid % NR
        cg = wid // NR
        col0 = cg * CW
        start_c = CPW * r

        pltpu.sync_copy(
            idx_hbm.at[pl.ds(start_c * CHUNK, CPW * CHUNK)], idx_v
        )

        @pl.loop(0, G)
        def _(row):
            @pl.loop(0, CW, step=16)
            def _(col):
                acc_v[row, pl.ds(col, 16)] = jnp.zeros((16,), jnp.float32)

        lane = lax.iota(jnp.int32, 16)

        @pl.loop(0, CPW)
        def _(i):
            chunk = start_c + i
            pltpu.sync_copy(
                x_hbm.at[pl.ds(chunk * CHUNK, CHUNK), pl.ds(col0, CW)],
                rows_v,
            )

            @pl.loop(0, CHUNK, step=16)
            def _(r0):
                idx16 = idx_v[pl.ds(i * CHUNK + r0, 16)]
                for j in range(16):
                    seg_vec = jnp.full((16,), idx16[j], jnp.int32)
                    vals = [
                        rows_v[r0 + j, pl.ds(16 * kk, 16)]
                        for kk in range(CW // 16)
                    ]
                    for kk in range(CW // 16):
                        plsc.addupdate_scatter(
                            acc_v,
                            [seg_vec, lane + (16 * kk)],
                            vals[kk],
                        )

        pltpu.sync_copy(acc_v, out_hbm.at[r, :, pl.ds(col0, CW)])

    return k(x, idx)


def _tc_tail(x_tail, idx_tail3d):
    def body(i_ref, x_ref, o_ref):
        step = pl.program_id(0)
        onehot = (
            i_ref[0, 0, :][:, None]
            == lax.broadcasted_iota(jnp.int32, (BT, G), 1)
        ).astype(jnp.float32)
        part = lax.dot_general(
            onehot,
            x_ref[...],
            (((0,), (0,)), ((), ())),
            preferred_element_type=jnp.float32,
            precision=lax.Precision.HIGHEST,
        )

        @pl.when(step == 0)
        def _():
            o_ref[...] = part

        @pl.when(step > 0)
        def _():
            o_ref[...] += part

    return pl.pallas_call(
        body,
        grid=(TB,),
        in_specs=[
            pl.BlockSpec((1, 1, BT), lambda i: (i, 0, 0)),
            pl.BlockSpec((BT, D), lambda i: (i, 0)),
        ],
        out_specs=pl.BlockSpec((G, D), lambda i: (0, 0)),
        out_shape=jax.ShapeDtypeStruct((G, D), jnp.float32),
    )(idx_tail3d, x_tail)


def _tc_head(parts, tail, W, b):
    def body(p_ref, t_ref, w_ref, b_ref, o_ref):
        acc = jnp.sum(p_ref[...], axis=0) + t_ref[...]
        logits = lax.dot_general(
            acc,
            w_ref[...],
            (((1,), (1,)), ((), ())),
            preferred_element_type=jnp.float32,
            precision=lax.Precision.HIGHEST,
        )
        logits = logits + b_ref[...]
        m = jnp.max(logits, axis=1, keepdims=True)
        lse = jnp.log(jnp.sum(jnp.exp(logits - m), axis=1, keepdims=True)) + m
        o_ref[...] = logits - lse

    return pl.pallas_call(
        body,
        out_shape=jax.ShapeDtypeStruct((G, NUM_LABELS), jnp.float32),
    )(parts, tail, W, b.reshape(1, NUM_LABELS))


def kernel(node_embedding_matrix, batch_x_index, W, b):
    idx = batch_x_index.astype(jnp.int32)
    parts = _sc_segment_sum(node_embedding_matrix, idx)

    x_tail = jnp.pad(node_embedding_matrix[SC_ROWS:], ((0, TAIL_PAD), (0, 0)))
    idx_tail = jnp.pad(idx[SC_ROWS:], (0, TAIL_PAD), constant_values=G)
    tail = _tc_tail(x_tail, idx_tail.reshape(TB, 1, BT))

    return _tc_head(parts, tail, W, b)


# speedup vs baseline: 1.4587x; 1.4587x over previous
import dataclasses
import functools

import jax
import jax.numpy as jnp
from jax import lax
from jax.experimental import pallas as pl
from jax.experimental.pallas import tpu as pltpu
from jax.experimental.pallas import tpu_sc as plsc

N_ROWS = 50000
D = 256
G = 512
NUM_LABELS = 10

NR = 16
NCOL = 2
CW = D // NCOL

CHUNK = 400
N_CHUNKS = N_ROWS // CHUNK
CPW = 5
SC_ROWS = NR * CPW * CHUNK

TAIL = N_ROWS - SC_ROWS
BT = CHUNK
TB = TAIL // BT


def _sc_segment_sum(x, idx):
    mesh = plsc.VectorSubcoreMesh(core_axis_name="c", subcore_axis_name="s")

    cp = pltpu.CompilerParams()
    if "needs_layout_passes" in pltpu.CompilerParams.__dataclass_fields__:
        cp = dataclasses.replace(cp, needs_layout_passes=False)

    @functools.partial(
        pl.kernel,
        compiler_params=cp,
        out_type=jax.ShapeDtypeStruct((NR, G, D), jnp.float32),
        mesh=mesh,
        scratch_types=[
            pltpu.VMEM((CHUNK, CW), jnp.float32),
            pltpu.VMEM((CPW * CHUNK,), jnp.int32),
            pltpu.VMEM((G, CW), jnp.float32),
        ],
    )
    def k(x_hbm, idx_hbm, out_hbm, rows_v, idx_v, acc_v):
        c = lax.axis_index("c")
        s = lax.axis_index("s")
        wid = s * 2 + c
        r = wid % NR
        cg = wid // NR
        col0 = cg * CW
        start_c = CPW * r

        pltpu.sync_copy(
            idx_hbm.at[pl.ds(start_c * CHUNK, CPW * CHUNK)], idx_v
        )

        @pl.loop(0, G)
        def _(row):
            @pl.loop(0, CW, step=16)
            def _(col):
                acc_v[row, pl.ds(col, 16)] = jnp.zeros((16,), jnp.float32)

        lane = lax.iota(jnp.int32, 16)

        @pl.loop(0, CPW)
        def _(i):
            chunk = start_c + i
            pltpu.sync_copy(
                x_hbm.at[pl.ds(chunk * CHUNK, CHUNK), pl.ds(col0, CW)],
                rows_v,
            )

            @pl.loop(0, CHUNK, step=16)
            def _(r0):
                idx16 = idx_v[pl.ds(i * CHUNK + r0, 16)]
                for j in range(16):
                    seg_vec = jnp.full((16,), idx16[j], jnp.int32)
                    vals = [
                        rows_v[r0 + j, pl.ds(16 * kk, 16)]
                        for kk in range(CW // 16)
                    ]
                    for kk in range(CW // 16):
                        plsc.addupdate_scatter(
                            acc_v,
                            [seg_vec, lane + (16 * kk)],
                            vals[kk],
                        )

        pltpu.sync_copy(acc_v, out_hbm.at[r, :, pl.ds(col0, CW)])

    return k(x, idx)


def _tc_tail(x_tail, idx_tail3d):
    def body(i_ref, x_ref, o_ref):
        step = pl.program_id(0)
        onehot = (
            i_ref[0, 0, :][:, None]
            == lax.broadcasted_iota(jnp.int32, (BT, G), 1)
        ).astype(jnp.bfloat16)
        xb = x_ref[...]
        hi = xb.astype(jnp.bfloat16)
        lo = (xb - hi.astype(jnp.float32)).astype(jnp.bfloat16)
        dims = (((0,), (0,)), ((), ()))
        part = lax.dot_general(
            onehot, hi, dims, preferred_element_type=jnp.float32
        ) + lax.dot_general(
            onehot, lo, dims, preferred_element_type=jnp.float32
        )

        @pl.when(step == 0)
        def _():
            o_ref[...] = part

        @pl.when(step > 0)
        def _():
            o_ref[...] += part

    return pl.pallas_call(
        body,
        grid=(TB,),
        in_specs=[
            pl.BlockSpec((1, 1, BT), lambda i: (SC_ROWS // BT + i, 0, 0)),
            pl.BlockSpec((BT, D), lambda i: (SC_ROWS // BT + i, 0)),
        ],
        out_specs=pl.BlockSpec((G, D), lambda i: (0, 0)),
        out_shape=jax.ShapeDtypeStruct((G, D), jnp.float32),
    )(idx_tail3d, x_tail)


def _tc_head(parts, tail, W, b):
    def body(p_ref, t_ref, w_ref, b_ref, o_ref):
        acc = jnp.sum(p_ref[...], axis=0) + t_ref[...]
        logits = lax.dot_general(
            acc,
            w_ref[...],
            (((1,), (1,)), ((), ())),
            preferred_element_type=jnp.float32,
            precision=lax.Precision.HIGHEST,
        )
        logits = logits + b_ref[...]
        m = jnp.max(logits, axis=1, keepdims=True)
        lse = jnp.log(jnp.sum(jnp.exp(logits - m), axis=1, keepdims=True)) + m
        o_ref[...] = logits - lse

    return pl.pallas_call(
        body,
        out_shape=jax.ShapeDtypeStruct((G, NUM_LABELS), jnp.float32),
    )(parts, tail, W, b.reshape(1, NUM_LABELS))


def kernel(node_embedding_matrix, batch_x_index, W, b):
    idx = batch_x_index.astype(jnp.int32)
    parts = _sc_segment_sum(node_embedding_matrix, idx)

    tail = _tc_tail(node_embedding_matrix, idx.reshape(N_CHUNKS, 1, CHUNK))

    return _tc_head(parts, tail, W, b)
